# baseline (speedup 1.0000x reference)
import functools

import jax
import jax.numpy as jnp
from jax import lax
from jax.experimental import pallas as pl
from jax.experimental.pallas import tpu as pltpu
from jax.experimental.pallas import tpu_sc as plsc

NC = 2
NS = 16
NW = NC * NS
L = 16

D = 128
E = 320000
EPW = E // NW
C = 80
NCHUNKS = EPW // C


@functools.lru_cache(maxsize=None)
def _build():
  mesh = plsc.VectorSubcoreMesh(core_axis_name="c", subcore_axis_name="s")

  @functools.partial(
      pl.kernel,
      mesh=mesh,
      compiler_params=pltpu.CompilerParams(needs_layout_passes=False),
      out_type=jax.ShapeDtypeStruct((E,), jnp.float32),
      scratch_types=[
          pltpu.VMEM((2 * EPW,), jnp.int32),
          pltpu.VMEM((EPW,), jnp.float32),
          pltpu.VMEM((2 * C, D), jnp.float32),
          pltpu.VMEM((2 * C, D), jnp.float32),
          pltpu.SemaphoreType.DMA,
          pltpu.SemaphoreType.DMA,
      ],
  )
  def sc_kernel(z_hbm, e_hbm, out_hbm, eidx, o_v, r0, r1, sem0, sem1):
    wid = lax.axis_index("s") * NC + lax.axis_index("c")
    wbase = wid * EPW
    lane = lax.iota(jnp.int32, L)
    fifteen = jnp.full((L,), L - 1, jnp.int32)

    pltpu.sync_copy(e_hbm.at[pl.ds(2 * wbase, 2 * EPW)], eidx)

    slots = ((r0, sem0), (r1, sem1))

    def issue(t, s):
      rows, sem = slots[s]
      pltpu.make_async_copy(
          z_hbm.at[eidx.at[pl.ds(t * 2 * C, 2 * C)]], rows, sem).start()

    def wait(t, s):
      rows, sem = slots[s]
      pltpu.make_async_copy(
          z_hbm.at[eidx.at[pl.ds(t * 2 * C, 2 * C)]], rows, sem).wait()

    def compute(t, s):
      rows, _ = slots[s]

      def group(g, carry2):

        def sub(sg, red):
          el0 = sg * 4
          for de in range(4):
            el = el0 + de
            e = g * L + el

            def dotpart(k):
              return (rows[2 * e, pl.ds(k * L, L)]
                      * rows[2 * e + 1, pl.ds(k * L, L)]
                      + rows[2 * e, pl.ds((k + 1) * L, L)]
                      * rows[2 * e + 1, pl.ds((k + 1) * L, L)])

            t0 = dotpart(0) + dotpart(2)
            t1 = dotpart(4) + dotpart(6)
            acc = t0 + t1
            tot = jnp.take_along_axis(plsc.cumsum(acc), fifteen, axis=0)
            red = jnp.where(lane == el, tot, red)
          return red

        red = lax.fori_loop(0, 4, sub, jnp.zeros((L,), jnp.float32))
        o_v[pl.ds(t * C + g * L, L)] = red
        return carry2

      lax.fori_loop(0, C // L, group, 0)

    def step(t, b, issue_next):
      if issue_next:

        @pl.when(t + 1 < NCHUNKS)
        def _():
          issue(t + 1, 1 - b)

      wait(t, b)
      compute(t, b)

    issue(0, 0)

    def outer(g, carry):
      for b in (0, 1):
        step(g * 2 + b, b, True)
      return carry

    lax.fori_loop(0, NCHUNKS // 2, outer, 0)
    if NCHUNKS % 2:
      step(NCHUNKS - 1, 0, False)

    pltpu.sync_copy(o_v, out_hbm.at[pl.ds(wbase, EPW)])

  return sc_kernel


def kernel(z, edges):
  return _build()(z, edges.reshape(-1))


# speedup vs baseline: 2.2765x; 2.2765x over previous
import functools

import jax
import jax.numpy as jnp
from jax import lax
from jax.experimental import pallas as pl
from jax.experimental.pallas import tpu as pltpu
from jax.experimental.pallas import tpu_sc as plsc

NC = 2
NS = 16
NW = NC * NS
L = 16

D = 128
E = 320000
EPW = E // NW
C = 80
NCHUNKS = EPW // C
NSLOT = 4


@functools.lru_cache(maxsize=None)
def _build():
  mesh = plsc.VectorSubcoreMesh(core_axis_name="c", subcore_axis_name="s")

  @functools.partial(
      pl.kernel,
      mesh=mesh,
      compiler_params=pltpu.CompilerParams(needs_layout_passes=False),
      out_type=jax.ShapeDtypeStruct((E,), jnp.float32),
      scratch_types=[
          pltpu.VMEM((EPW,), jnp.int32),
          pltpu.VMEM((EPW,), jnp.int32),
          pltpu.VMEM((EPW,), jnp.float32),
          pltpu.VMEM((C, D), jnp.float32),
          pltpu.VMEM((C, D), jnp.float32),
          pltpu.VMEM((C, D), jnp.float32),
          pltpu.VMEM((C, D), jnp.float32),
          pltpu.VMEM((C, D), jnp.float32),
          pltpu.VMEM((C, D), jnp.float32),
          pltpu.VMEM((C, D), jnp.float32),
          pltpu.VMEM((C, D), jnp.float32),
          pltpu.SemaphoreType.DMA,
          pltpu.SemaphoreType.DMA,
          pltpu.SemaphoreType.DMA,
          pltpu.SemaphoreType.DMA,
      ],
  )
  def sc_kernel(z_hbm, a_hbm, b_hbm, out_hbm,
                aidx, bidx, o_v, u0, v0, u1, v1, u2, v2, u3, v3,
                sem0, sem1, sem2, sem3):
    wid = lax.axis_index("s") * NC + lax.axis_index("c")
    wbase = wid * EPW
    lane = lax.iota(jnp.int32, L)
    fifteen = jnp.full((L,), L - 1, jnp.int32)

    pltpu.sync_copy(a_hbm.at[pl.ds(wbase, EPW)], aidx)
    pltpu.sync_copy(b_hbm.at[pl.ds(wbase, EPW)], bidx)

    slots = ((u0, v0, sem0), (u1, v1, sem1), (u2, v2, sem2), (u3, v3, sem3))

    def issue(t, s):
      u_v, v_v, sem = slots[s]
      pltpu.make_async_copy(
          z_hbm.at[aidx.at[pl.ds(t * C, C)]], u_v, sem).start()
      pltpu.make_async_copy(
          z_hbm.at[bidx.at[pl.ds(t * C, C)]], v_v, sem).start()

    def wait(t, s):
      u_v, v_v, sem = slots[s]
      pltpu.make_async_copy(
          z_hbm.at[aidx.at[pl.ds(t * C, C)]], u_v, sem).wait()
      pltpu.make_async_copy(
          z_hbm.at[bidx.at[pl.ds(t * C, C)]], v_v, sem).wait()

    def compute(t, s):
      u_v, v_v, _ = slots[s]

      def group(g, carry2):

        def sub(sg, red):
          el0 = sg * 4
          for de in range(4):
            el = el0 + de
            e = g * L + el

            def dotpart(k):
              return (u_v[e, pl.ds(k * L, L)] * v_v[e, pl.ds(k * L, L)]
                      + u_v[e, pl.ds((k + 1) * L, L)]
                      * v_v[e, pl.ds((k + 1) * L, L)])

            t0 = dotpart(0) + dotpart(2)
            t1 = dotpart(4) + dotpart(6)
            acc = t0 + t1
            tot = jnp.take_along_axis(plsc.cumsum(acc), fifteen, axis=0)
            red = jnp.where(lane == el, tot, red)
          return red

        red = lax.fori_loop(0, 4, sub, jnp.zeros((L,), jnp.float32))
        o_v[pl.ds(t * C + g * L, L)] = red
        return carry2

      lax.fori_loop(0, C // L, group, 0)

    def step(t, b, issue_next):
      if issue_next:

        @pl.when(t + NSLOT - 1 < NCHUNKS)
        def _():
          issue(t + NSLOT - 1, (b + NSLOT - 1) % NSLOT)

      wait(t, b)
      compute(t, b)

    for p in range(NSLOT - 1):
      issue(p, p)

    def outer(g, carry):
      for b in range(NSLOT):
        step(g * NSLOT + b, b, True)
      return carry

    lax.fori_loop(0, NCHUNKS // NSLOT, outer, 0)
    for t in range(NCHUNKS - NCHUNKS % NSLOT, NCHUNKS):
      step(t, t % NSLOT, True)

    pltpu.sync_copy(o_v, out_hbm.at[pl.ds(wbase, EPW)])

  return sc_kernel


def kernel(z, edges):
  a = edges[:, 0]
  b = edges[:, 1]
  return _build()(z, a, b)
